# baseline (speedup 1.0000x reference)
import jax
import jax.numpy as jnp
from jax.experimental import pallas as pl
from jax.experimental.pallas import tpu as pltpu
from jax.experimental.pallas import tpu_sc as plsc

VOCABN = 16384
DIMN = 1024
NH = 16
NG = 4
HPGN = 4
DKN = 64
DVN = 64
LCMP = 32
DSTR = 16
LSELN = 32
NSELN = 4
WWIN = 256
HIDN = 4096
SEQ = 2048

QT = 256
NCMP = (SEQ - LCMP) // DSTR + 1
NBLK = SEQ // LSELN
SCALE = 1.0 / (DKN ** 0.5)
GATHER_W = 128
GSPLIT = 4
GDIM = DIMN // GSPLIT
GN = SEQ * GSPLIT


def _embed_gather(idx, embed):
    mesh = plsc.VectorSubcoreMesh(core_axis_name="core",
                                  subcore_axis_name="subcore")
    embed4 = embed.reshape(VOCABN * GSPLIT, GDIM)
    idx4 = (idx.reshape(SEQ, 1) * GSPLIT
            + jnp.arange(GSPLIT, dtype=jnp.int32).reshape(1, GSPLIT)
            ).reshape(1, GN)

    @pl.kernel(out_type=jax.ShapeDtypeStruct((GN, GDIM), jnp.float32),
               mesh=mesh)
    def gk(x_hbm, i_hbm, o_hbm):
        def body(i_vmem, o_vmem):
            pltpu.sync_copy(x_hbm.at[i_vmem.at[0]], o_vmem)

        pltpu.emit_pipeline(
            body,
            grid=(GN // GATHER_W,),
            in_specs=[pl.BlockSpec((1, GATHER_W), index_map=lambda i: (0, i))],
            out_specs=[pl.BlockSpec((GATHER_W, GDIM),
                                    index_map=lambda i: (i, 0))],
            core_axis_name=("core", "subcore"),
            dimension_semantics=(pltpu.PARALLEL,),
        )(i_hbm, o_hbm)

    return gk(embed4, idx4).reshape(SEQ, DIMN)


def _bdot(a, b, dims):
    return jax.lax.dot_general(a, b, dims, preferred_element_type=jnp.float32)


def _bdot16(a, b16, dims):
    return jax.lax.dot_general(a.astype(jnp.bfloat16), b16, dims,
                               preferred_element_type=jnp.float32)


def _rope_all(mat, nheads, cos, sin):
    parts = []
    for h in range(nheads):
        x1 = mat[:, h * DKN:h * DKN + DKN // 2]
        x2 = mat[:, h * DKN + DKN // 2:(h + 1) * DKN]
        parts.append(x1 * cos - x2 * sin)
        parts.append(x1 * sin + x2 * cos)
    return jnp.concatenate(parts, axis=1)


def _proj_body(x_ref, ln1_ref, wq_ref, wk_ref, wv_ref, wg_ref, cos_ref,
               sin_ref, q_ref, k_ref, v_ref, g_ref):
    x = x_ref[...]
    ms = jnp.mean(x * x, axis=-1, keepdims=True)
    xn = x * jax.lax.rsqrt(ms + 1e-6) * ln1_ref[...]
    cos = cos_ref[...]
    sin = sin_ref[...]

    q = _bdot16(xn, wq_ref[...], (((1,), (0,)), ((), ())))
    k = _bdot16(xn, wk_ref[...], (((1,), (0,)), ((), ())))
    v = _bdot16(xn, wv_ref[...], (((1,), (0,)), ((), ())))
    g = _bdot16(xn, wg_ref[...], (((1,), (0,)), ((), ())))
    q_ref[...] = _rope_all(q, NH, cos, sin)
    k_ref[...] = _rope_all(k, NG, cos, sin)
    v_ref[...] = v
    g_ref[...] = jax.nn.sigmoid(g)


def _proj(x, ln1, Wq, Wk, Wv, Wg, cos, sin):
    nst = SEQ // QT
    return pl.pallas_call(
        _proj_body,
        grid=(nst,),
        in_specs=[
            pl.BlockSpec((QT, DIMN), lambda i: (i, 0)),
            pl.BlockSpec((1, DIMN), lambda i: (0, 0)),
            pl.BlockSpec((DIMN, NH * DKN), lambda i: (0, 0)),
            pl.BlockSpec((DIMN, NG * DKN), lambda i: (0, 0)),
            pl.BlockSpec((DIMN, NG * DVN), lambda i: (0, 0)),
            pl.BlockSpec((DIMN, NH * 3), lambda i: (0, 0)),
            pl.BlockSpec((QT, DKN // 2), lambda i: (i, 0)),
            pl.BlockSpec((QT, DKN // 2), lambda i: (i, 0)),
        ],
        out_specs=[
            pl.BlockSpec((QT, NH * DKN), lambda i: (i, 0)),
            pl.BlockSpec((QT, NG * DKN), lambda i: (i, 0)),
            pl.BlockSpec((QT, NG * DVN), lambda i: (i, 0)),
            pl.BlockSpec((QT, NH * 3), lambda i: (i, 0)),
        ],
        out_shape=[
            jax.ShapeDtypeStruct((SEQ, NH * DKN), jnp.float32),
            jax.ShapeDtypeStruct((SEQ, NG * DKN), jnp.float32),
            jax.ShapeDtypeStruct((SEQ, NG * DVN), jnp.float32),
            jax.ShapeDtypeStruct((SEQ, NH * 3), jnp.float32),
        ],
    )(x, ln1, Wq, Wk, Wv, Wg, cos, sin)


def _attn_body(q_ref, k_ref, v_ref, g_ref, o_ref):
    i = pl.program_id(0)
    NEG = -1e30

    srow_i = i * QT + jax.lax.broadcasted_iota(jnp.int32, (QT, 1), 0)

    start = jnp.maximum(i - 1, 0) * QT

    out_chunks = []
    for g in range(NG):
        kg = k_ref[:, g * DKN:(g + 1) * DKN]
        vg = v_ref[:, g * DVN:(g + 1) * DVN]

        p16k = jnp.mean(kg.reshape(SEQ // DSTR, DSTR, DKN), axis=1)
        p16v = jnp.mean(vg.reshape(SEQ // DSTR, DSTR, DVN), axis=1)
        k_cmp = (p16k[:NCMP] + p16k[1:NCMP + 1]) * 0.5
        v_cmp = (p16v[:NCMP] + p16v[1:NCMP + 1]) * 0.5

        ncol = jax.lax.broadcasted_iota(jnp.int32, (QT, NCMP), 1)
        mcmp = (DSTR * ncol + LCMP - 1) <= srow_i

        o_cmp = []
        phsum = jnp.zeros((QT, NCMP), jnp.float32)
        for h in range(HPGN):
            hh = g * HPGN + h
            qh = q_ref[:, hh * DKN:(hh + 1) * DKN]
            sc = _bdot(
                qh, k_cmp, (((1,), (1,)), ((), ()))) * SCALE
            sc = jnp.where(mcmp, sc, NEG)
            mmax = jnp.max(sc, axis=-1, keepdims=True)
            e = jnp.exp(sc - mmax) * mcmp.astype(jnp.float32)
            p = e / jnp.maximum(jnp.sum(e, axis=-1, keepdims=True), 1e-9)
            phsum = phsum + p
            o_cmp.append(_bdot(
                p, v_cmp, (((1,), (0,)), ((), ()))))

        nrow = jax.lax.broadcasted_iota(jnp.int32, (NCMP, NBLK), 0)
        mcol = jax.lax.broadcasted_iota(jnp.int32, (NCMP, NBLK), 1)
        fold = ((nrow // 2) == mcol).astype(jnp.float32)
        imp = _bdot(phsum, fold, (((1,), (0,)), ((), ())))

        bcol = jax.lax.broadcasted_iota(jnp.int32, (QT, NBLK), 1)
        cur = srow_i // LSELN
        bonus = (jnp.where(bcol == cur, 1e6, 0.0)
                 + jnp.where(bcol == 0, 1e6, 0.0))
        valid = (bcol * LSELN) <= srow_i
        impv = jnp.where(valid, imp + bonus, -jnp.inf)

        selmask = jnp.zeros((QT, NBLK), jnp.bool_)
        for _ in range(NSELN):
            mx = jnp.max(impv, axis=-1, keepdims=True)
            cand = jnp.where(impv == mx, bcol, NBLK + 1)
            first = jnp.min(cand, axis=-1, keepdims=True)
            hit = bcol == first
            selmask = selmask | hit
            impv = jnp.where(hit, -jnp.inf, impv)

        erow = jax.lax.broadcasted_iota(jnp.int32, (NBLK, SEQ), 0)
        ecol = jax.lax.broadcasted_iota(jnp.int32, (NBLK, SEQ), 1)
        expand = ((ecol // LSELN) == erow).astype(jnp.float32)
        tmask = _bdot(
            selmask.astype(jnp.float32), expand, (((1,), (0,)), ((), ())))

        tcol = jax.lax.broadcasted_iota(jnp.int32, (QT, SEQ), 1)
        causal = tcol <= srow_i
        selm = (tmask > 0.5) & causal
        selm_f = selm.astype(jnp.float32)

        kwin = k_ref[pl.ds(start, 2 * QT), g * DKN:(g + 1) * DKN]
        vwin = v_ref[pl.ds(start, 2 * QT), g * DVN:(g + 1) * DVN]
        twin = start + jax.lax.broadcasted_iota(jnp.int32, (QT, 2 * QT), 1)
        winm = (twin <= srow_i) & ((srow_i - twin) < WWIN)

        for h in range(HPGN):
            hh = g * HPGN + h
            qh = q_ref[:, hh * DKN:(hh + 1) * DKN]

            sc = _bdot(
                qh, kg, (((1,), (1,)), ((), ()))) * SCALE
            sc = jnp.where(selm, sc, NEG)
            mmax = jnp.max(sc, axis=-1, keepdims=True)
            e = jnp.exp(sc - mmax) * selm_f
            p = e / jnp.maximum(jnp.sum(e, axis=-1, keepdims=True), 1e-9)
            o_sel = _bdot(p, vg, (((1,), (0,)), ((), ())))

            sw = _bdot(
                qh, kwin, (((1,), (1,)), ((), ()))) * SCALE
            sw = jnp.where(winm, sw, NEG)
            wmax = jnp.max(sw, axis=-1, keepdims=True)
            ew = jnp.exp(sw - wmax)
            pw = ew / jnp.sum(ew, axis=-1, keepdims=True)
            o_w = _bdot(pw, vwin, (((1,), (0,)), ((), ())))

            g0 = g_ref[:, 3 * hh:3 * hh + 1]
            g1 = g_ref[:, 3 * hh + 1:3 * hh + 2]
            g2 = g_ref[:, 3 * hh + 2:3 * hh + 3]
            out_chunks.append(g0 * o_cmp[h] + g1 * o_sel + g2 * o_w)

    o_ref[...] = jnp.concatenate(out_chunks, axis=1)


def _attention(q_r, k_r, v_p, gates):
    nst = SEQ // QT
    return pl.pallas_call(
        _attn_body,
        grid=(nst,),
        in_specs=[
            pl.BlockSpec((QT, NH * DKN), lambda i: (i, 0)),
            pl.BlockSpec((SEQ, NG * DKN), lambda i: (0, 0)),
            pl.BlockSpec((SEQ, NG * DVN), lambda i: (0, 0)),
            pl.BlockSpec((QT, NH * 3), lambda i: (i, 0)),
        ],
        out_specs=pl.BlockSpec((QT, NH * DVN), lambda i: (i, 0)),
        out_shape=jax.ShapeDtypeStruct((SEQ, NH * DVN), jnp.float32),
    )(q_r, k_r, v_p, gates)


def _oproj_body(a_ref, x_ref, wo_ref, ln2_ref, h_ref, hn_ref):
    h = x_ref[...] + _bdot16(
        a_ref[...], wo_ref[...], (((1,), (0,)), ((), ())))
    h_ref[...] = h
    ms = jnp.mean(h * h, axis=-1, keepdims=True)
    hn_ref[...] = h * jax.lax.rsqrt(ms + 1e-6) * ln2_ref[...]


def _oproj(att, x, Wo, ln2):
    nst = SEQ // QT
    return pl.pallas_call(
        _oproj_body,
        grid=(nst,),
        in_specs=[
            pl.BlockSpec((QT, NH * DVN), lambda i: (i, 0)),
            pl.BlockSpec((QT, DIMN), lambda i: (i, 0)),
            pl.BlockSpec((NH * DVN, DIMN), lambda i: (0, 0)),
            pl.BlockSpec((1, DIMN), lambda i: (0, 0)),
        ],
        out_specs=[
            pl.BlockSpec((QT, DIMN), lambda i: (i, 0)),
            pl.BlockSpec((QT, DIMN), lambda i: (i, 0)),
        ],
        out_shape=[
            jax.ShapeDtypeStruct((SEQ, DIMN), jnp.float32),
            jax.ShapeDtypeStruct((SEQ, DIMN), jnp.float32),
        ],
    )(att, x, Wo, ln2)


HT = 512


def _mlp_body(hn_ref, h_ref, w1_ref, w3_ref, w2_ref, o_ref):
    j = pl.program_id(0)
    hn = hn_ref[...]
    u = jax.nn.silu(_bdot16(
        hn, w1_ref[...], (((1,), (0,)), ((), ())))) * _bdot16(
        hn, w3_ref[...], (((1,), (0,)), ((), ())))
    contrib = _bdot16(u, w2_ref[...], (((1,), (0,)), ((), ())))

    @pl.when(j == 0)
    def _():
        o_ref[...] = h_ref[...] + contrib

    @pl.when(j > 0)
    def _():
        o_ref[...] = o_ref[...] + contrib


def _mlp(hn, h1, w1, w3, w2):
    return pl.pallas_call(
        _mlp_body,
        grid=(HIDN // HT,),
        in_specs=[
            pl.BlockSpec((SEQ, DIMN), lambda j: (0, 0)),
            pl.BlockSpec((SEQ, DIMN), lambda j: (0, 0)),
            pl.BlockSpec((DIMN, HT), lambda j: (0, j)),
            pl.BlockSpec((DIMN, HT), lambda j: (0, j)),
            pl.BlockSpec((HT, DIMN), lambda j: (j, 0)),
        ],
        out_specs=pl.BlockSpec((SEQ, DIMN), lambda j: (0, 0)),
        out_shape=jax.ShapeDtypeStruct((SEQ, DIMN), jnp.float32),
    )(hn, h1, w1, w3, w2)


VT = 1024


def _lmhead_body(h_ref, w_ref, o_ref):
    o_ref[...] = _bdot16(
        h_ref[...], w_ref[...], (((1,), (0,)), ((), ())))


def _lmhead(h2, lm_head):
    return pl.pallas_call(
        _lmhead_body,
        grid=(VOCABN // VT,),
        in_specs=[
            pl.BlockSpec((SEQ, DIMN), lambda j: (0, 0)),
            pl.BlockSpec((DIMN, VT), lambda j: (0, j)),
        ],
        out_specs=pl.BlockSpec((SEQ, VT), lambda j: (0, j)),
        out_shape=jax.ShapeDtypeStruct((SEQ, VOCABN), jnp.float32),
    )(h2, lm_head)


def kernel(x_tok, embed, Wq, Wk, Wv, Wg, Wo, w1, w2, w3, lm_head, ln1, ln2):
    idx = x_tok.reshape(1, SEQ).astype(jnp.int32)
    x = _embed_gather(idx, embed)
    pos = jnp.arange(SEQ, dtype=jnp.float32)
    jf = jnp.arange(DKN // 2, dtype=jnp.float32)
    freqs = 1.0 / (10000.0 ** (jf / (DKN // 2)))
    ang = pos[:, None] * freqs[None, :]
    cos = jnp.cos(ang)
    sin = jnp.sin(ang)
    bf = jnp.bfloat16
    q_r, k_r, v_p, gates = _proj(x, ln1.reshape(1, DIMN), Wq.astype(bf),
                                 Wk.astype(bf), Wv.astype(bf), Wg.astype(bf),
                                 cos, sin)
    att = _attention(q_r, k_r, v_p, gates)
    h1, hn = _oproj(att, x, Wo.astype(bf), ln2.reshape(1, DIMN))
    h2 = _mlp(hn, h1, w1.astype(bf), w3.astype(bf), w2.astype(bf))
    logits = _lmhead(h2.astype(bf), lm_head.astype(bf))
    return logits.reshape(1, SEQ, VOCABN)


# speedup vs baseline: 1.3143x; 1.3143x over previous
import jax
import jax.numpy as jnp
from jax.experimental import pallas as pl
from jax.experimental.pallas import tpu as pltpu
from jax.experimental.pallas import tpu_sc as plsc

VOCABN = 16384
DIMN = 1024
NH = 16
NG = 4
HPGN = 4
DKN = 64
DVN = 64
LCMP = 32
DSTR = 16
LSELN = 32
NSELN = 4
WWIN = 256
HIDN = 4096
SEQ = 2048

QT = 256
NCMP = (SEQ - LCMP) // DSTR + 1
NBLK = SEQ // LSELN
SCALE = 1.0 / (DKN ** 0.5)
GATHER_W = 128
GSPLIT = 4
GDIM = DIMN // GSPLIT
GN = SEQ * GSPLIT


def _embed_gather(idx, embed):
    mesh = plsc.VectorSubcoreMesh(core_axis_name="core",
                                  subcore_axis_name="subcore")
    embed4 = embed.reshape(VOCABN * GSPLIT, GDIM)
    idx4 = (idx.reshape(SEQ, 1) * GSPLIT
            + jnp.arange(GSPLIT, dtype=jnp.int32).reshape(1, GSPLIT)
            ).reshape(1, GN)

    @pl.kernel(out_type=jax.ShapeDtypeStruct((GN, GDIM), jnp.float32),
               mesh=mesh)
    def gk(x_hbm, i_hbm, o_hbm):
        def body(i_vmem, o_vmem):
            pltpu.sync_copy(x_hbm.at[i_vmem.at[0]], o_vmem)

        pltpu.emit_pipeline(
            body,
            grid=(GN // GATHER_W,),
            in_specs=[pl.BlockSpec((1, GATHER_W), index_map=lambda i: (0, i))],
            out_specs=[pl.BlockSpec((GATHER_W, GDIM),
                                    index_map=lambda i: (i, 0))],
            core_axis_name=("core", "subcore"),
            dimension_semantics=(pltpu.PARALLEL,),
        )(i_hbm, o_hbm)

    return gk(embed4, idx4).reshape(SEQ, DIMN)


def _bdot(a, b, dims):
    return jax.lax.dot_general(a, b, dims, preferred_element_type=jnp.float32)


def _rope_all(mat, nheads, cos, sin):
    parts = []
    for h in range(nheads):
        x1 = mat[:, h * DKN:h * DKN + DKN // 2]
        x2 = mat[:, h * DKN + DKN // 2:(h + 1) * DKN]
        parts.append(x1 * cos - x2 * sin)
        parts.append(x1 * sin + x2 * cos)
    return jnp.concatenate(parts, axis=1)


def _proj_body(x_ref, ln1_ref, wq_ref, wk_ref, wv_ref, wg_ref, cos_ref,
               sin_ref, q_ref, k_ref, v_ref, g_ref):
    x = x_ref[...]
    ms = jnp.mean(x * x, axis=-1, keepdims=True)
    xn = x * jax.lax.rsqrt(ms + 1e-6) * ln1_ref[...]
    cos = cos_ref[...]
    sin = sin_ref[...]

    q = _bdot(xn, wq_ref[...], (((1,), (0,)), ((), ())))
    k = _bdot(xn, wk_ref[...], (((1,), (0,)), ((), ())))
    v = _bdot(xn, wv_ref[...], (((1,), (0,)), ((), ())))
    g = _bdot(xn, wg_ref[...], (((1,), (0,)), ((), ())))
    q_ref[...] = _rope_all(q, NH, cos, sin)
    k_ref[...] = _rope_all(k, NG, cos, sin)
    v_ref[...] = v
    g_ref[...] = jax.nn.sigmoid(g)


def _proj(x, ln1, Wq, Wk, Wv, Wg, cos, sin):
    nst = SEQ // QT
    return pl.pallas_call(
        _proj_body,
        grid=(nst,),
        in_specs=[
            pl.BlockSpec((QT, DIMN), lambda i: (i, 0)),
            pl.BlockSpec((1, DIMN), lambda i: (0, 0)),
            pl.BlockSpec((DIMN, NH * DKN), lambda i: (0, 0)),
            pl.BlockSpec((DIMN, NG * DKN), lambda i: (0, 0)),
            pl.BlockSpec((DIMN, NG * DVN), lambda i: (0, 0)),
            pl.BlockSpec((DIMN, NH * 3), lambda i: (0, 0)),
            pl.BlockSpec((QT, DKN // 2), lambda i: (i, 0)),
            pl.BlockSpec((QT, DKN // 2), lambda i: (i, 0)),
        ],
        out_specs=[
            pl.BlockSpec((QT, NH * DKN), lambda i: (i, 0)),
            pl.BlockSpec((QT, NG * DKN), lambda i: (i, 0)),
            pl.BlockSpec((QT, NG * DVN), lambda i: (i, 0)),
            pl.BlockSpec((QT, NH * 3), lambda i: (i, 0)),
        ],
        out_shape=[
            jax.ShapeDtypeStruct((SEQ, NH * DKN), jnp.float32),
            jax.ShapeDtypeStruct((SEQ, NG * DKN), jnp.float32),
            jax.ShapeDtypeStruct((SEQ, NG * DVN), jnp.float32),
            jax.ShapeDtypeStruct((SEQ, NH * 3), jnp.float32),
        ],
    )(x, ln1, Wq, Wk, Wv, Wg, cos, sin)


def _attn_body(q_ref, k_ref, v_ref, g_ref, o_ref):
    i = pl.program_id(0)
    NEG = -1e30

    srow_i = i * QT + jax.lax.broadcasted_iota(jnp.int32, (QT, 1), 0)

    start = jnp.maximum(i - 1, 0) * QT

    out_chunks = []
    for g in range(NG):
        kg = k_ref[:, g * DKN:(g + 1) * DKN]
        vg = v_ref[:, g * DVN:(g + 1) * DVN]

        p16k = jnp.mean(kg.reshape(SEQ // DSTR, DSTR, DKN), axis=1)
        p16v = jnp.mean(vg.reshape(SEQ // DSTR, DSTR, DVN), axis=1)
        k_cmp = (p16k[:NCMP] + p16k[1:NCMP + 1]) * 0.5
        v_cmp = (p16v[:NCMP] + p16v[1:NCMP + 1]) * 0.5

        ncol = jax.lax.broadcasted_iota(jnp.int32, (QT, NCMP), 1)
        mcmp = (DSTR * ncol + LCMP - 1) <= srow_i

        o_cmp = []
        phsum = jnp.zeros((QT, NCMP), jnp.float32)
        for h in range(HPGN):
            hh = g * HPGN + h
            qh = q_ref[:, hh * DKN:(hh + 1) * DKN]
            sc = _bdot(
                qh, k_cmp, (((1,), (1,)), ((), ()))) * SCALE
            sc = jnp.where(mcmp, sc, NEG)
            mmax = jnp.max(sc, axis=-1, keepdims=True)
            e = jnp.exp(sc - mmax) * mcmp.astype(jnp.float32)
            p = e / jnp.maximum(jnp.sum(e, axis=-1, keepdims=True), 1e-9)
            phsum = phsum + p
            o_cmp.append(_bdot(
                p, v_cmp, (((1,), (0,)), ((), ()))))

        nrow = jax.lax.broadcasted_iota(jnp.int32, (NCMP, NBLK), 0)
        mcol = jax.lax.broadcasted_iota(jnp.int32, (NCMP, NBLK), 1)
        fold = ((nrow // 2) == mcol).astype(jnp.float32)
        imp = _bdot(phsum, fold, (((1,), (0,)), ((), ())))

        bcol = jax.lax.broadcasted_iota(jnp.int32, (QT, NBLK), 1)
        cur = srow_i // LSELN
        bonus = (jnp.where(bcol == cur, 1e6, 0.0)
                 + jnp.where(bcol == 0, 1e6, 0.0))
        valid = (bcol * LSELN) <= srow_i
        impv = jnp.where(valid, imp + bonus, -jnp.inf)

        selmask = jnp.zeros((QT, NBLK), jnp.bool_)
        for _ in range(NSELN):
            mx = jnp.max(impv, axis=-1, keepdims=True)
            cand = jnp.where(impv == mx, bcol, NBLK + 1)
            first = jnp.min(cand, axis=-1, keepdims=True)
            hit = bcol == first
            selmask = selmask | hit
            impv = jnp.where(hit, -jnp.inf, impv)

        selmask_f = selmask.astype(jnp.float32)
        erow = jax.lax.broadcasted_iota(jnp.int32, (NBLK, QT), 0)
        ecolb = jax.lax.broadcasted_iota(jnp.int32, (NBLK, QT), 1) // LSELN
        tcol0 = jax.lax.broadcasted_iota(jnp.int32, (QT, QT), 1)
        qhs = [q_ref[:, (g * HPGN + h) * DKN:(g * HPGN + h + 1) * DKN]
               for h in range(HPGN)]

        def tbody(t, accs):
            kt = k_ref[pl.ds(t * QT, QT), g * DKN:(g + 1) * DKN]
            vt = v_ref[pl.ds(t * QT, QT), g * DVN:(g + 1) * DVN]
            vt_ext = jnp.concatenate(
                [vt, jnp.ones((QT, 1), jnp.float32)], axis=1)
            e_t = (erow == (t * (QT // LSELN) + ecolb)).astype(jnp.float32)
            tmask = _bdot(selmask_f, e_t, (((1,), (0,)), ((), ())))
            tcol = t * QT + tcol0
            madd = jnp.where((tmask > 0.5) & (tcol <= srow_i), 0.0, NEG)
            new = []
            for h in range(HPGN):
                sc = _bdot(qhs[h], kt, (((1,), (1,)), ((), ()))) * SCALE
                e = jnp.exp(sc + madd)
                new.append(accs[h] + _bdot(e, vt_ext, (((1,), (0,)), ((), ()))))
            return tuple(new)

        accs0 = tuple(jnp.zeros((QT, DVN + 1), jnp.float32)
                      for _ in range(HPGN))
        accs = jax.lax.fori_loop(0, i + 1, tbody, accs0)

        kwin = k_ref[pl.ds(start, 2 * QT), g * DKN:(g + 1) * DKN]
        vwin = v_ref[pl.ds(start, 2 * QT), g * DVN:(g + 1) * DVN]
        vwin_ext = jnp.concatenate(
            [vwin, jnp.ones((2 * QT, 1), jnp.float32)], axis=1)
        twin = start + jax.lax.broadcasted_iota(jnp.int32, (QT, 2 * QT), 1)
        wadd = jnp.where((twin <= srow_i) & ((srow_i - twin) < WWIN),
                         0.0, NEG)

        for h in range(HPGN):
            hh = g * HPGN + h

            a = accs[h]
            o_sel = a[:, :DVN] / jnp.maximum(a[:, DVN:DVN + 1], 1e-9)

            sw = _bdot(qhs[h], kwin, (((1,), (1,)), ((), ()))) * SCALE
            ew = jnp.exp(sw + wadd)
            aw = _bdot(ew, vwin_ext, (((1,), (0,)), ((), ())))
            o_w = aw[:, :DVN] / aw[:, DVN:DVN + 1]

            g0 = g_ref[:, 3 * hh:3 * hh + 1]
            g1 = g_ref[:, 3 * hh + 1:3 * hh + 2]
            g2 = g_ref[:, 3 * hh + 2:3 * hh + 3]
            out_chunks.append(g0 * o_cmp[h] + g1 * o_sel + g2 * o_w)

    o_ref[...] = jnp.concatenate(out_chunks, axis=1)


def _attention(q_r, k_r, v_p, gates):
    nst = SEQ // QT
    return pl.pallas_call(
        _attn_body,
        grid=(nst,),
        in_specs=[
            pl.BlockSpec((QT, NH * DKN), lambda i: (i, 0)),
            pl.BlockSpec((SEQ, NG * DKN), lambda i: (0, 0)),
            pl.BlockSpec((SEQ, NG * DVN), lambda i: (0, 0)),
            pl.BlockSpec((QT, NH * 3), lambda i: (i, 0)),
        ],
        out_specs=pl.BlockSpec((QT, NH * DVN), lambda i: (i, 0)),
        out_shape=jax.ShapeDtypeStruct((SEQ, NH * DVN), jnp.float32),
    )(q_r, k_r, v_p, gates)


def _oproj_body(a_ref, x_ref, wo_ref, ln2_ref, h_ref, hn_ref):
    h = x_ref[...] + _bdot(
        a_ref[...], wo_ref[...], (((1,), (0,)), ((), ())))
    h_ref[...] = h
    ms = jnp.mean(h * h, axis=-1, keepdims=True)
    hn_ref[...] = h * jax.lax.rsqrt(ms + 1e-6) * ln2_ref[...]


def _oproj(att, x, Wo, ln2):
    nst = SEQ // QT
    return pl.pallas_call(
        _oproj_body,
        grid=(nst,),
        in_specs=[
            pl.BlockSpec((QT, NH * DVN), lambda i: (i, 0)),
            pl.BlockSpec((QT, DIMN), lambda i: (i, 0)),
            pl.BlockSpec((NH * DVN, DIMN), lambda i: (0, 0)),
            pl.BlockSpec((1, DIMN), lambda i: (0, 0)),
        ],
        out_specs=[
            pl.BlockSpec((QT, DIMN), lambda i: (i, 0)),
            pl.BlockSpec((QT, DIMN), lambda i: (i, 0)),
        ],
        out_shape=[
            jax.ShapeDtypeStruct((SEQ, DIMN), jnp.float32),
            jax.ShapeDtypeStruct((SEQ, DIMN), jnp.float32),
        ],
    )(att, x, Wo, ln2)


HT = 512


def _mlp_body(hn_ref, h_ref, w1_ref, w3_ref, w2_ref, o_ref):
    j = pl.program_id(0)
    hn = hn_ref[...]
    u = jax.nn.silu(_bdot(
        hn, w1_ref[...], (((1,), (0,)), ((), ())))) * _bdot(
        hn, w3_ref[...], (((1,), (0,)), ((), ())))
    contrib = _bdot(u, w2_ref[...], (((1,), (0,)), ((), ())))

    @pl.when(j == 0)
    def _():
        o_ref[...] = h_ref[...] + contrib

    @pl.when(j > 0)
    def _():
        o_ref[...] = o_ref[...] + contrib


def _mlp(hn, h1, w1, w3, w2):
    return pl.pallas_call(
        _mlp_body,
        grid=(HIDN // HT,),
        in_specs=[
            pl.BlockSpec((SEQ, DIMN), lambda j: (0, 0)),
            pl.BlockSpec((SEQ, DIMN), lambda j: (0, 0)),
            pl.BlockSpec((DIMN, HT), lambda j: (0, j)),
            pl.BlockSpec((DIMN, HT), lambda j: (0, j)),
            pl.BlockSpec((HT, DIMN), lambda j: (j, 0)),
        ],
        out_specs=pl.BlockSpec((SEQ, DIMN), lambda j: (0, 0)),
        out_shape=jax.ShapeDtypeStruct((SEQ, DIMN), jnp.float32),
    )(hn, h1, w1, w3, w2)


VT = 1024


def _lmhead_body(h_ref, w_ref, o_ref):
    o_ref[...] = _bdot(
        h_ref[...], w_ref[...], (((1,), (0,)), ((), ())))


def _lmhead(h2, lm_head):
    return pl.pallas_call(
        _lmhead_body,
        grid=(VOCABN // VT,),
        in_specs=[
            pl.BlockSpec((SEQ, DIMN), lambda j: (0, 0)),
            pl.BlockSpec((DIMN, VT), lambda j: (0, j)),
        ],
        out_specs=pl.BlockSpec((SEQ, VT), lambda j: (0, j)),
        out_shape=jax.ShapeDtypeStruct((SEQ, VOCABN), jnp.float32),
    )(h2, lm_head)


def kernel(x_tok, embed, Wq, Wk, Wv, Wg, Wo, w1, w2, w3, lm_head, ln1, ln2):
    idx = x_tok.reshape(1, SEQ).astype(jnp.int32)
    x = _embed_gather(idx, embed)
    pos = jnp.arange(SEQ, dtype=jnp.float32)
    jf = jnp.arange(DKN // 2, dtype=jnp.float32)
    freqs = 1.0 / (10000.0 ** (jf / (DKN // 2)))
    ang = pos[:, None] * freqs[None, :]
    cos = jnp.cos(ang)
    sin = jnp.sin(ang)
    q_r, k_r, v_p, gates = _proj(x, ln1.reshape(1, DIMN), Wq, Wk, Wv, Wg,
                                 cos, sin)
    att = _attention(q_r, k_r, v_p, gates)
    h1, hn = _oproj(att, x, Wo, ln2.reshape(1, DIMN))
    h2 = _mlp(hn, h1, w1, w3, w2)
    logits = _lmhead(h2, lm_head)
    return logits.reshape(1, SEQ, VOCABN)
